# baseline (speedup 1.0000x reference)
import functools

import numpy as np
import jax
import jax.numpy as jnp
from jax import lax
from jax.experimental import pallas as pl
from jax.experimental.pallas import tpu as pltpu
from jax.experimental.pallas import tpu_sc as plsc

PRIME = (1 << 31) - 1
DIM = 32
N_HASH = 2
BATCH = 16384
B_ROWS = 1_000_000
W_SIZE = 125_000

_rng = np.random.RandomState(1234)
_A0 = _rng.randint(1, PRIME, size=N_HASH)
_C0 = _rng.randint(0, PRIME, size=N_HASH)
_A1 = _rng.randint(1, PRIME, size=N_HASH)
_C1 = _rng.randint(0, PRIME, size=N_HASH)

NC, NS, L = 2, 16, 16
NW = NC * NS
BPW = BATCH // NW
G = BPW // L
IDX_C = 128
NCHUNK = BPW // IDX_C

_M16 = 0xFFFF
_M15 = 0x7FFF
_M31 = 0x7FFFFFFF


def _mod_p(v):
    r = (v >> jnp.uint32(31)) + (v & jnp.uint32(_M31))
    return jnp.where(r >= jnp.uint32(PRIME), r - jnp.uint32(PRIME), r)


def _hash16(x0, x1, a, c, r_range):
    a = int(a)
    c = int(c)
    a0 = a & _M16
    a1 = a >> 16
    t1 = x1 * jnp.uint32(2 * a1)
    m = x1 * jnp.uint32(a0) + x0 * jnp.uint32(a1)
    t2 = (m >> jnp.uint32(15)) + ((m & jnp.uint32(_M15)) << jnp.uint32(16))
    t3 = x0 * jnp.uint32(a0)
    s1 = _mod_p(t1 + jnp.uint32(c))
    u = _mod_p(_mod_p(t2) + _mod_p(t3))
    h = _mod_p(u + s1)
    hi = h.astype(jnp.int32)
    q = (hi.astype(jnp.float32) * np.float32(1.0 / r_range)).astype(jnp.int32)
    r = hi - q * jnp.int32(r_range)
    r = jnp.where(r < 0, r + jnp.int32(r_range), r)
    r = jnp.where(r >= jnp.int32(r_range), r - jnp.int32(r_range), r)
    return r


_mesh = plsc.VectorSubcoreMesh(core_axis_name="c", subcore_axis_name="s")


@functools.partial(
    pl.kernel,
    mesh=_mesh,
    out_type=jax.ShapeDtypeStruct((BATCH, DIM), jnp.float32),
    compiler_params=pltpu.CompilerParams(
        needs_layout_passes=False, use_tc_tiling_on_sc=False),
    scratch_types=[
        pltpu.VMEM((BPW,), jnp.int32),
        pltpu.VMEM((NCHUNK, IDX_C), jnp.int32),
        pltpu.VMEM((NCHUNK, IDX_C), jnp.int32),
        pltpu.VMEM((NCHUNK, IDX_C), jnp.int32),
        pltpu.VMEM((NCHUNK, IDX_C), jnp.int32),
        pltpu.VMEM((BPW, DIM), jnp.float32),
        pltpu.VMEM((BPW, DIM), jnp.float32),
        pltpu.VMEM((BPW,), jnp.float32),
        pltpu.VMEM((BPW,), jnp.float32),
        pltpu.VMEM((BPW, DIM), jnp.float32),
        pltpu.SemaphoreType.DMA,
        pltpu.SemaphoreType.DMA,
        pltpu.SemaphoreType.DMA,
        pltpu.SemaphoreType.DMA,
    ],
)
def _hash_embed(x_hbm, table_hbm, w_hbm, out_hbm,
                x_v, ia_v, ib_v, iwa_v, iwb_v,
                rows_a, rows_b, w_a, w_b, out_v,
                sem_a, sem_b, sem_wa, sem_wb):
    wid = lax.axis_index("s") * jnp.int32(NC) + lax.axis_index("c")
    base = wid * jnp.int32(BPW)
    pltpu.sync_copy(x_hbm.at[pl.ds(base, BPW)], x_v)

    def hash_body(g, carry):
        xv = x_v[pl.ds(g * jnp.int32(L), L)].astype(jnp.uint32)
        x0 = xv & jnp.uint32(_M16)
        x1 = xv >> jnp.uint32(16)
        row = g >> jnp.int32(3)
        col = (g & jnp.int32(7)) * jnp.int32(L)
        ia_v[row, pl.ds(col, L)] = _hash16(x0, x1, _A0[0], _C0[0], B_ROWS)
        ib_v[row, pl.ds(col, L)] = _hash16(x0, x1, _A0[1], _C0[1], B_ROWS)
        iwa_v[row, pl.ds(col, L)] = _hash16(x0, x1, _A1[0], _C1[0], W_SIZE)
        iwb_v[row, pl.ds(col, L)] = _hash16(x0, x1, _A1[1], _C1[1], W_SIZE)
        return carry

    lax.fori_loop(jnp.int32(0), jnp.int32(G), hash_body, jnp.int32(0))

    copies = []
    for j in range(NCHUNK):
        jj = jnp.int32(j)
        sl = pl.ds(jnp.int32(j * IDX_C), IDX_C)
        copies.append(pltpu.async_copy(
            table_hbm.at[ia_v.at[jj]], rows_a.at[sl], sem_a))
        copies.append(pltpu.async_copy(
            table_hbm.at[ib_v.at[jj]], rows_b.at[sl], sem_b))
        copies.append(pltpu.async_copy(
            w_hbm.at[iwa_v.at[jj]], w_a.at[sl], sem_wa))
        copies.append(pltpu.async_copy(
            w_hbm.at[iwb_v.at[jj]], w_b.at[sl], sem_wb))
    for c in copies:
        c.wait()

    def comb_body(b, carry):
        bb = jnp.full((L,), b, jnp.int32)
        wa = plsc.load_gather(w_a, [bb])
        wb = plsc.load_gather(w_b, [bb])
        a0 = rows_a[b, pl.ds(0, L)]
        a1 = rows_a[b, pl.ds(L, L)]
        b0 = rows_b[b, pl.ds(0, L)]
        b1 = rows_b[b, pl.ds(L, L)]
        out_v[b, pl.ds(0, L)] = wa * a0 + wb * b0
        out_v[b, pl.ds(L, L)] = wa * a1 + wb * b1
        return carry

    lax.fori_loop(jnp.int32(0), jnp.int32(BPW), comb_body, jnp.int32(0))

    pltpu.sync_copy(out_v, out_hbm.at[pl.ds(base, BPW)])


def kernel(x, table, weights):
    return _hash_embed(x.astype(jnp.int32), table, weights)
